# baseline (speedup 1.0000x reference)
import functools

import jax
import jax.numpy as jnp
from jax import lax
from jax.experimental import pallas as pl
from jax.experimental.pallas import tpu as pltpu
from jax.experimental.pallas import tpu_sc as plsc

BATCH = 16384
EMBED_DIM = 16
NUM_CORES = 2
NUM_SUBCORES = 16
NW = NUM_CORES * NUM_SUBCORES
BPW = BATCH // NW


def _sc_partial_sums(input_user, input_item, pred_data_label,
                     user_tab, item_tab, bias_tab):
  mesh = plsc.VectorSubcoreMesh(core_axis_name="c", subcore_axis_name="s")

  @functools.partial(
      pl.kernel,
      out_type=[
          jax.ShapeDtypeStruct((NW, EMBED_DIM), jnp.float32),
          jax.ShapeDtypeStruct((NW, EMBED_DIM), jnp.float32),
      ],
      mesh=mesh,
      compiler_params=pltpu.CompilerParams(use_tc_tiling_on_sc=False),
      scratch_types=[
          pltpu.VMEM((BPW,), jnp.int32),
          pltpu.VMEM((BPW,), jnp.int32),
          pltpu.VMEM((BPW,), jnp.int32),
          pltpu.VMEM((BPW, EMBED_DIM), jnp.float32),
          pltpu.VMEM((BPW, EMBED_DIM), jnp.float32),
          pltpu.VMEM((BPW, EMBED_DIM), jnp.float32),
          pltpu.VMEM((BPW,), jnp.float32),
          pltpu.VMEM((BPW,), jnp.float32),
          pltpu.VMEM((EMBED_DIM,), jnp.float32),
          pltpu.SemaphoreType.DMA,
          pltpu.SemaphoreType.DMA,
          pltpu.SemaphoreType.DMA,
          pltpu.SemaphoreType.DMA,
          pltpu.SemaphoreType.DMA,
      ],
  )
  def sc_kernel(uidx_hbm, iidx_hbm, gidx_hbm, utab_hbm, itab_hbm, btab_hbm,
                out_s1, out_s2,
                idx_u, idx_i, idx_g, rows_u, rows_i, rows_g,
                bias_i, bias_g, acc_st,
                sem_u, sem_i, sem_g, sem_bi, sem_bg):
    wid = lax.axis_index("s") * NUM_CORES + lax.axis_index("c")
    base = wid * BPW

    pltpu.sync_copy(uidx_hbm.at[pl.ds(base, BPW)], idx_u)
    pltpu.sync_copy(iidx_hbm.at[pl.ds(base, BPW)], idx_i)
    pltpu.sync_copy(gidx_hbm.at[pl.ds(base, BPW)], idx_g)

    cu = pltpu.async_copy(utab_hbm.at[idx_u], rows_u, sem_u)
    ci = pltpu.async_copy(itab_hbm.at[idx_i], rows_i, sem_i)
    cg = pltpu.async_copy(itab_hbm.at[idx_g], rows_g, sem_g)
    cbi = pltpu.async_copy(btab_hbm.at[idx_i], bias_i, sem_bi)
    cbg = pltpu.async_copy(btab_hbm.at[idx_g], bias_g, sem_bg)
    cu.wait()
    ci.wait()
    cg.wait()
    cbi.wait()
    cbg.wait()

    zero = jnp.zeros((EMBED_DIM,), jnp.float32)

    def dot_body(j, carry):
      a1, a2 = carry
      u = rows_u[j]
      return a1 + u * rows_i[j], a2 + u * rows_g[j]

    a1, a2 = lax.fori_loop(0, BPW, dot_body, (zero, zero))

    def bias_body(t, carry):
      b1, b2 = carry
      return (b1 + bias_i[pl.ds(t * EMBED_DIM, EMBED_DIM)],
              b2 + bias_g[pl.ds(t * EMBED_DIM, EMBED_DIM)])

    b1, b2 = lax.fori_loop(0, BPW // EMBED_DIM, bias_body, (zero, zero))

    acc_st[...] = a1 + b1
    pltpu.sync_copy(acc_st, out_s1.at[wid])
    acc_st[...] = a2 + b2
    pltpu.sync_copy(acc_st, out_s2.at[wid])

  return sc_kernel(input_user, input_item, pred_data_label,
                   user_tab, item_tab, bias_tab)


def _tc_loss(s1_partials, s2_partials):
  def body(s1_ref, s2_ref, out_ref):
    inv_b = 1.0 / float(BATCH)
    s1 = jnp.sum(s1_ref[...]) * inv_b
    s2 = jnp.sum(s2_ref[...]) * inv_b

    def softplus(x):
      return jnp.maximum(x, 0.0) + jnp.log(1.0 + jnp.exp(-jnp.abs(x)))

    out_ref[...] = jnp.full((1, 1), softplus(-s2) + softplus(s1))

  out = pl.pallas_call(
      body,
      out_shape=jax.ShapeDtypeStruct((1, 1), jnp.float32),
  )(s1_partials, s2_partials)
  return out[0, 0]


def kernel(input_user, input_item, pred_data_label,
           D_user_embeddings, D_item_embeddings, D_item_bias):
  s1, s2 = _sc_partial_sums(input_user, input_item, pred_data_label,
                            D_user_embeddings, D_item_embeddings, D_item_bias)
  return _tc_loss(s1, s2)
